# baseline (speedup 1.0000x reference)
import functools

import jax
import jax.numpy as jnp
from jax import lax
from jax.experimental import pallas as pl
from jax.experimental.pallas import tpu as pltpu
from jax.experimental.pallas import tpu_sc as plsc

K = 8192
D = 768
BN = 4608

TK = 2048
TBM = 768
TBS = 768
TBA = 512
TR = 768


def _rownorm(x):
    return x / jnp.sqrt(jnp.sum(x * x, axis=1, keepdims=True) + 1e-12)


def _sim_exp_kernel(mem_ref, proj_ref, e_ref, re_ref, mn_s, pnt_s):
    k = pl.program_id(0)
    j = pl.program_id(1)

    @pl.when(j == 0)
    def _():
        mn_s[...] = _rownorm(mem_ref[...]).astype(jnp.bfloat16)

    @pl.when(k == 0)
    def _():
        pnt_s[:, pl.ds(j * TBM, TBM)] = jnp.transpose(
            _rownorm(proj_ref[...])).astype(jnp.bfloat16)

    mn = mn_s[...]
    CH = 256
    rs = None
    for c in range(TBM // CH):
        sim = lax.dot_general(mn, pnt_s[:, pl.ds(j * TBM + c * CH, CH)],
                              (((1,), (0,)), ((), ())),
                              preferred_element_type=jnp.float32)
        e = jnp.exp(sim / 0.05)
        e_ref[:, c * CH:(c + 1) * CH] = e
        rc = jnp.transpose(jnp.sum(e, axis=1, keepdims=True))
        rs = rc if rs is None else rs + rc

    @pl.when(j == 0)
    def _():
        re_ref[...] = rs

    @pl.when(j > 0)
    def _():
        re_ref[...] += rs


def _sink_pass_kernel(r_in_ref, e_ref, r_out_ref):
    i = pl.program_id(0)
    e = e_ref[...]
    u = jnp.transpose(1.0 / (jnp.float32(K) * r_in_ref[...]))
    c = jnp.sum(e * u, axis=0, keepdims=True)
    v = 1.0 / (jnp.float32(BN) * c)
    rp = jnp.transpose(jnp.sum(e * v, axis=1, keepdims=True))

    @pl.when(i == 0)
    def _():
        r_out_ref[...] = rp

    @pl.when(i > 0)
    def _():
        r_out_ref[...] += rp


def _argmax_kernel(r_in_ref, e_ref, idx_ref):
    u = jnp.transpose(1.0 / (jnp.float32(K) * r_in_ref[...]))
    s = e_ref[...] * u
    m = jnp.max(s, axis=0, keepdims=True)
    ii = lax.broadcasted_iota(jnp.int32, (K, TBA), 0)
    cand = jnp.where(s == m, ii, K)
    idx_ref[...] = jnp.min(cand, axis=0, keepdims=True)


def _glu_kernel(ma_ref, w_ref, b_ref, proj_ref, out_ref):
    a = ma_ref[...].astype(jnp.bfloat16)
    gate = lax.dot_general(a, w_ref[:, D:], (((1,), (0,)), ((), ())),
                           preferred_element_type=jnp.float32) + b_ref[:, D:]
    sig = 1.0 / (1.0 + jnp.exp(-gate))
    lin = lax.dot_general(a, w_ref[:, :D], (((1,), (0,)), ((), ())),
                          preferred_element_type=jnp.float32) + b_ref[:, :D]
    out_ref[...] = (proj_ref[...] + lin * sig) * 0.5


def _make_sc_gather():
    info = plsc.get_sparse_core_info()
    nc, ns = info.num_cores, info.num_subcores
    nw = nc * ns
    bpw = BN // nw
    half = bpw // 2
    mesh = plsc.VectorSubcoreMesh(core_axis_name="c", subcore_axis_name="s")

    @functools.partial(
        pl.kernel,
        mesh=mesh,
        out_type=jax.ShapeDtypeStruct((BN, D), jnp.float32),
        scratch_types=[
            pltpu.VMEM((2, half), jnp.int32),
            pltpu.VMEM((bpw, D), jnp.float32),
            pltpu.SemaphoreType.DMA,
        ],
    )
    def gather_rows(table_hbm, idx_hbm, out_hbm, idx_v, rows_v, sem):
        wid = lax.axis_index("s") * nc + lax.axis_index("c")
        base = wid * bpw
        pltpu.sync_copy(idx_hbm.at[pl.ds(base, half)], idx_v.at[0])
        pltpu.sync_copy(idx_hbm.at[pl.ds(base + half, half)], idx_v.at[1])
        cp0 = pltpu.async_copy(table_hbm.at[idx_v.at[0]],
                               rows_v.at[pl.ds(0, half)], sem)
        cp1 = pltpu.async_copy(table_hbm.at[idx_v.at[1]],
                               rows_v.at[pl.ds(half, half)], sem)
        cp0.wait()
        cp1.wait()
        pltpu.sync_copy(rows_v, out_hbm.at[pl.ds(base, bpw)])

    return gather_rows


def kernel(projections, memory, W, b):
    Bdim, N, d = projections.shape
    proj2d = projections.reshape(-1, d)

    e_arr, r1 = pl.pallas_call(
        _sim_exp_kernel,
        grid=(K // TK, BN // TBM),
        in_specs=[
            pl.BlockSpec((TK, D), lambda k, j: (k, 0)),
            pl.BlockSpec((TBM, D), lambda k, j: (jnp.where(k == 0, j, 0), 0)),
        ],
        out_specs=[
            pl.BlockSpec((TK, TBM), lambda k, j: (k, j)),
            pl.BlockSpec((1, TK), lambda k, j: (0, k)),
        ],
        out_shape=[
            jax.ShapeDtypeStruct((K, BN), jnp.float32),
            jax.ShapeDtypeStruct((1, K), jnp.float32),
        ],
        scratch_shapes=[
            pltpu.VMEM((TK, D), jnp.bfloat16),
            pltpu.VMEM((D, BN), jnp.bfloat16),
        ],
    )(memory, proj2d)

    sink = pl.pallas_call(
        _sink_pass_kernel,
        grid=(BN // TBS,),
        in_specs=[
            pl.BlockSpec((1, K), lambda i: (0, 0)),
            pl.BlockSpec((K, TBS), lambda i: (0, i)),
        ],
        out_specs=pl.BlockSpec((1, K), lambda i: (0, 0)),
        out_shape=jax.ShapeDtypeStruct((1, K), jnp.float32),
    )
    r2 = sink(r1, e_arr)
    r3 = sink(r2, e_arr)

    idx2d = pl.pallas_call(
        _argmax_kernel,
        grid=(BN // TBA,),
        in_specs=[
            pl.BlockSpec((1, K), lambda i: (0, 0)),
            pl.BlockSpec((K, TBA), lambda i: (0, i)),
        ],
        out_specs=pl.BlockSpec((1, TBA), lambda i: (0, i)),
        out_shape=jax.ShapeDtypeStruct((1, BN), jnp.int32),
    )(r3, e_arr)
    idx = idx2d.reshape(BN)

    mem_assign = _make_sc_gather()(memory, idx)

    out2d = pl.pallas_call(
        _glu_kernel,
        grid=(BN // TR,),
        in_specs=[
            pl.BlockSpec((TR, D), lambda i: (i, 0)),
            pl.BlockSpec((D, 2 * D), lambda i: (0, 0)),
            pl.BlockSpec((1, 2 * D), lambda i: (0, 0)),
            pl.BlockSpec((TR, D), lambda i: (i, 0)),
        ],
        out_specs=pl.BlockSpec((TR, D), lambda i: (i, 0)),
        out_shape=jax.ShapeDtypeStruct((BN, D), jnp.float32),
    )(mem_assign, W.astype(jnp.bfloat16), b.reshape(1, 2 * D), proj2d)

    return out2d.reshape(Bdim, N, d)


# speedup vs baseline: 1.0469x; 1.0469x over previous
import functools

import jax
import jax.numpy as jnp
from jax import lax
from jax.experimental import pallas as pl
from jax.experimental.pallas import tpu as pltpu
from jax.experimental.pallas import tpu_sc as plsc

K = 8192
D = 768
BN = 4608

TK = 2048
TBM = 512
TBS = 512
TR = 768


def _rownorm(x):
    return x / jnp.sqrt(jnp.sum(x * x, axis=1, keepdims=True) + 1e-12)


def _sim_exp_kernel(mem_ref, proj_ref, e_ref, re_ref, mn_s, pnt_s):
    k = pl.program_id(0)
    j = pl.program_id(1)

    @pl.when(j == 0)
    def _():
        mn_s[...] = _rownorm(mem_ref[...]).astype(jnp.bfloat16)

    @pl.when(k == 0)
    def _():
        pnt_s[:, pl.ds(j * TBM, TBM)] = jnp.transpose(
            _rownorm(proj_ref[...])).astype(jnp.bfloat16)

    mn = mn_s[...]
    CH = 256
    rs = None
    for c in range(TBM // CH):
        sim = lax.dot_general(mn, pnt_s[:, pl.ds(j * TBM + c * CH, CH)],
                              (((1,), (0,)), ((), ())),
                              preferred_element_type=jnp.float32)
        e = jnp.exp(sim / 0.05)
        e_ref[0, :, c * CH:(c + 1) * CH] = e
        rc = jnp.sum(e, axis=1, keepdims=True)
        rs = rc if rs is None else rs + rc

    @pl.when(j == 0)
    def _():
        re_ref[...] = rs

    @pl.when(j > 0)
    def _():
        re_ref[...] += rs


def _sink_pass_kernel(r_in_ref, e_ref, r_out_ref):
    i = pl.program_id(0)
    e = e_ref[0]
    u = 1.0 / (jnp.float32(K) * r_in_ref[...])
    c = jnp.sum(e * u, axis=0, keepdims=True)
    v = 1.0 / (jnp.float32(BN) * c)
    rp = jnp.sum(e * v, axis=1, keepdims=True)

    @pl.when(i == 0)
    def _():
        r_out_ref[...] = rp

    @pl.when(i > 0)
    def _():
        r_out_ref[...] += rp


def _argmax_kernel(r_in_ref, e_ref, idx_ref):
    u = 1.0 / (jnp.float32(K) * r_in_ref[...])
    s = e_ref[0] * u
    m = jnp.max(s, axis=0, keepdims=True)
    ii = lax.broadcasted_iota(jnp.int32, (K, TBS), 0)
    cand = jnp.where(s == m, ii, K)
    idx_ref[...] = jnp.min(cand, axis=0, keepdims=True)


def _glu_kernel(ma_ref, w_ref, b_ref, proj_ref, out_ref):
    a = ma_ref[...].astype(jnp.bfloat16)
    gate = lax.dot_general(a, w_ref[:, D:], (((1,), (0,)), ((), ())),
                           preferred_element_type=jnp.float32) + b_ref[:, D:]
    sig = 1.0 / (1.0 + jnp.exp(-gate))
    lin = lax.dot_general(a, w_ref[:, :D], (((1,), (0,)), ((), ())),
                          preferred_element_type=jnp.float32) + b_ref[:, :D]
    out_ref[...] = (proj_ref[...] + lin * sig) * 0.5


def _make_sc_gather():
    info = plsc.get_sparse_core_info()
    nc, ns = info.num_cores, info.num_subcores
    nw = nc * ns
    bpw = BN // nw
    half = bpw // 2
    mesh = plsc.VectorSubcoreMesh(core_axis_name="c", subcore_axis_name="s")

    @functools.partial(
        pl.kernel,
        mesh=mesh,
        out_type=jax.ShapeDtypeStruct((BN, D), jnp.float32),
        scratch_types=[
            pltpu.VMEM((2, half), jnp.int32),
            pltpu.VMEM((bpw, D), jnp.float32),
            pltpu.SemaphoreType.DMA,
        ],
    )
    def gather_rows(table_hbm, idx_hbm, out_hbm, idx_v, rows_v, sem):
        wid = lax.axis_index("s") * nc + lax.axis_index("c")
        base = wid * bpw
        pltpu.sync_copy(idx_hbm.at[pl.ds(base, half)], idx_v.at[0])
        pltpu.sync_copy(idx_hbm.at[pl.ds(base + half, half)], idx_v.at[1])
        cp0 = pltpu.async_copy(table_hbm.at[idx_v.at[0]],
                               rows_v.at[pl.ds(0, half)], sem)
        cp1 = pltpu.async_copy(table_hbm.at[idx_v.at[1]],
                               rows_v.at[pl.ds(half, half)], sem)
        cp0.wait()
        cp1.wait()
        pltpu.sync_copy(rows_v, out_hbm.at[pl.ds(base, bpw)])

    return gather_rows


def kernel(projections, memory, W, b):
    Bdim, N, d = projections.shape
    proj2d = projections.reshape(-1, d)

    e_arr, r1 = pl.pallas_call(
        _sim_exp_kernel,
        grid=(K // TK, BN // TBM),
        in_specs=[
            pl.BlockSpec((TK, D), lambda k, j: (k, 0)),
            pl.BlockSpec((TBM, D), lambda k, j: (jnp.where(k == 0, j, 0), 0)),
        ],
        out_specs=[
            pl.BlockSpec((1, TK, TBM), lambda k, j: (j, k, 0)),
            pl.BlockSpec((TK, 1), lambda k, j: (k, 0)),
        ],
        out_shape=[
            jax.ShapeDtypeStruct((BN // TBM, K, TBM), jnp.float32),
            jax.ShapeDtypeStruct((K, 1), jnp.float32),
        ],
        scratch_shapes=[
            pltpu.VMEM((TK, D), jnp.bfloat16),
            pltpu.VMEM((D, BN), jnp.bfloat16),
        ],
    )(memory, proj2d)

    sink = pl.pallas_call(
        _sink_pass_kernel,
        grid=(BN // TBS,),
        in_specs=[
            pl.BlockSpec((K, 1), lambda i: (0, 0)),
            pl.BlockSpec((1, K, TBS), lambda i: (i, 0, 0)),
        ],
        out_specs=pl.BlockSpec((K, 1), lambda i: (0, 0)),
        out_shape=jax.ShapeDtypeStruct((K, 1), jnp.float32),
    )
    r2 = sink(r1, e_arr)
    r3 = sink(r2, e_arr)

    idx2d = pl.pallas_call(
        _argmax_kernel,
        grid=(BN // TBS,),
        in_specs=[
            pl.BlockSpec((K, 1), lambda i: (0, 0)),
            pl.BlockSpec((1, K, TBS), lambda i: (i, 0, 0)),
        ],
        out_specs=pl.BlockSpec((1, TBS), lambda i: (0, i)),
        out_shape=jax.ShapeDtypeStruct((1, BN), jnp.int32),
    )(r3, e_arr)
    idx = idx2d.reshape(BN)

    mem_assign = _make_sc_gather()(memory, idx)

    out2d = pl.pallas_call(
        _glu_kernel,
        grid=(BN // TR,),
        in_specs=[
            pl.BlockSpec((TR, D), lambda i: (i, 0)),
            pl.BlockSpec((D, 2 * D), lambda i: (0, 0)),
            pl.BlockSpec((1, 2 * D), lambda i: (0, 0)),
            pl.BlockSpec((TR, D), lambda i: (i, 0)),
        ],
        out_specs=pl.BlockSpec((TR, D), lambda i: (i, 0)),
        out_shape=jax.ShapeDtypeStruct((BN, D), jnp.float32),
    )(mem_assign, W.astype(jnp.bfloat16), b.reshape(1, 2 * D), proj2d)

    return out2d.reshape(Bdim, N, d)


# speedup vs baseline: 1.0684x; 1.0205x over previous
import functools

import jax
import jax.numpy as jnp
from jax import lax
from jax.experimental import pallas as pl
from jax.experimental.pallas import tpu as pltpu
from jax.experimental.pallas import tpu_sc as plsc

K = 8192
D = 768
BN = 4608

TK = 2048
TBM = 768
TBS = 512
TR = 768


def _rownorm(x):
    return x / jnp.sqrt(jnp.sum(x * x, axis=1, keepdims=True) + 1e-12)


def _sim_exp_kernel(mem_ref, proj_ref, e_ref, re_ref, mn_s, pnt_s):
    k = pl.program_id(0)
    j = pl.program_id(1)

    @pl.when(j == 0)
    def _():
        mn_s[...] = _rownorm(mem_ref[...]).astype(jnp.bfloat16)

    @pl.when(k == 0)
    def _():
        pnt_s[:, pl.ds(j * TBM, TBM)] = jnp.transpose(
            _rownorm(proj_ref[...])).astype(jnp.bfloat16)

    mn = mn_s[...]
    CH = 256
    rs = None
    for c in range(TBM // CH):
        sim = lax.dot_general(mn, pnt_s[:, pl.ds(j * TBM + c * CH, CH)],
                              (((1,), (0,)), ((), ())),
                              preferred_element_type=jnp.float32)
        e = jnp.exp(sim / 0.05)
        e_ref[:, c * CH:(c + 1) * CH] = e
        rc = jnp.sum(e, axis=1, keepdims=True)
        rs = rc if rs is None else rs + rc

    @pl.when(j == 0)
    def _():
        re_ref[...] = rs

    @pl.when(j > 0)
    def _():
        re_ref[...] += rs


def _sink_pass_kernel(r_in_ref, e_ref, r_out_ref):
    i = pl.program_id(0)
    e = e_ref[...]
    u = 1.0 / (jnp.float32(K) * r_in_ref[...])
    c = jnp.sum(e * u, axis=0, keepdims=True)
    v = 1.0 / (jnp.float32(BN) * c)
    rp = jnp.sum(e * v, axis=1, keepdims=True)

    @pl.when(i == 0)
    def _():
        r_out_ref[...] = rp

    @pl.when(i > 0)
    def _():
        r_out_ref[...] += rp


def _argmax_kernel(r_in_ref, e_ref, idx_ref):
    u = 1.0 / (jnp.float32(K) * r_in_ref[...])
    s = e_ref[...] * u
    m = jnp.max(s, axis=0, keepdims=True)
    ii = lax.broadcasted_iota(jnp.int32, (K, TBS), 0)
    cand = jnp.where(s == m, ii, K)
    idx_ref[...] = jnp.min(cand, axis=0, keepdims=True)


def _glu_kernel(ma_ref, w_ref, b_ref, proj_ref, out_ref):
    a = ma_ref[...].astype(jnp.bfloat16)
    gate = lax.dot_general(a, w_ref[:, D:], (((1,), (0,)), ((), ())),
                           preferred_element_type=jnp.float32) + b_ref[:, D:]
    sig = 1.0 / (1.0 + jnp.exp(-gate))
    lin = lax.dot_general(a, w_ref[:, :D], (((1,), (0,)), ((), ())),
                          preferred_element_type=jnp.float32) + b_ref[:, :D]
    out_ref[...] = (proj_ref[...] + lin * sig) * 0.5


def _make_sc_gather():
    info = plsc.get_sparse_core_info()
    nc, ns = info.num_cores, info.num_subcores
    nw = nc * ns
    bpw = BN // nw
    half = bpw // 2
    mesh = plsc.VectorSubcoreMesh(core_axis_name="c", subcore_axis_name="s")

    @functools.partial(
        pl.kernel,
        mesh=mesh,
        out_type=jax.ShapeDtypeStruct((BN, D), jnp.float32),
        scratch_types=[
            pltpu.VMEM((2, half), jnp.int32),
            pltpu.VMEM((bpw, D), jnp.float32),
            pltpu.SemaphoreType.DMA,
        ],
    )
    def gather_rows(table_hbm, idx_hbm, out_hbm, idx_v, rows_v, sem):
        wid = lax.axis_index("s") * nc + lax.axis_index("c")
        base = wid * bpw
        pltpu.sync_copy(idx_hbm.at[pl.ds(base, half)], idx_v.at[0])
        pltpu.sync_copy(idx_hbm.at[pl.ds(base + half, half)], idx_v.at[1])
        cp0 = pltpu.async_copy(table_hbm.at[idx_v.at[0]],
                               rows_v.at[pl.ds(0, half)], sem)
        cp1 = pltpu.async_copy(table_hbm.at[idx_v.at[1]],
                               rows_v.at[pl.ds(half, half)], sem)
        cp0.wait()
        cp1.wait()
        pltpu.sync_copy(rows_v, out_hbm.at[pl.ds(base, bpw)])

    return gather_rows


def kernel(projections, memory, W, b):
    Bdim, N, d = projections.shape
    proj2d = projections.reshape(-1, d)

    e_arr, r1 = pl.pallas_call(
        _sim_exp_kernel,
        grid=(K // TK, BN // TBM),
        in_specs=[
            pl.BlockSpec((TK, D), lambda k, j: (k, 0)),
            pl.BlockSpec((TBM, D), lambda k, j: (jnp.where(k == 0, j, 0), 0)),
        ],
        out_specs=[
            pl.BlockSpec((TK, TBM), lambda k, j: (k, j)),
            pl.BlockSpec((TK, 1), lambda k, j: (k, 0)),
        ],
        out_shape=[
            jax.ShapeDtypeStruct((K, BN), jnp.float32),
            jax.ShapeDtypeStruct((K, 1), jnp.float32),
        ],
        scratch_shapes=[
            pltpu.VMEM((TK, D), jnp.bfloat16),
            pltpu.VMEM((D, BN), jnp.bfloat16),
        ],
    )(memory, proj2d)

    sink = pl.pallas_call(
        _sink_pass_kernel,
        grid=(BN // TBS,),
        in_specs=[
            pl.BlockSpec((K, 1), lambda i: (0, 0)),
            pl.BlockSpec((K, TBS), lambda i: (0, i)),
        ],
        out_specs=pl.BlockSpec((K, 1), lambda i: (0, 0)),
        out_shape=jax.ShapeDtypeStruct((K, 1), jnp.float32),
    )
    r2 = sink(r1, e_arr)
    r3 = sink(r2, e_arr)

    idx2d = pl.pallas_call(
        _argmax_kernel,
        grid=(BN // TBS,),
        in_specs=[
            pl.BlockSpec((K, 1), lambda i: (0, 0)),
            pl.BlockSpec((K, TBS), lambda i: (0, i)),
        ],
        out_specs=pl.BlockSpec((1, TBS), lambda i: (0, i)),
        out_shape=jax.ShapeDtypeStruct((1, BN), jnp.int32),
    )(r3, e_arr)
    idx = idx2d.reshape(BN)

    mem_assign = _make_sc_gather()(memory, idx)

    out2d = pl.pallas_call(
        _glu_kernel,
        grid=(BN // TR,),
        in_specs=[
            pl.BlockSpec((TR, D), lambda i: (i, 0)),
            pl.BlockSpec((D, 2 * D), lambda i: (0, 0)),
            pl.BlockSpec((1, 2 * D), lambda i: (0, 0)),
            pl.BlockSpec((TR, D), lambda i: (i, 0)),
        ],
        out_specs=pl.BlockSpec((TR, D), lambda i: (i, 0)),
        out_shape=jax.ShapeDtypeStruct((BN, D), jnp.float32),
    )(mem_assign, W.astype(jnp.bfloat16), b.reshape(1, 2 * D), proj2d)

    return out2d.reshape(Bdim, N, d)
